# Initial kernel scaffold; baseline (speedup 1.0000x reference)
import functools

import jax
import jax.numpy as jnp
from jax import lax
from jax.experimental import pallas as pl
from jax.experimental.pallas import tpu as pltpu
from jax.experimental.pallas import tpu_sc as plsc

NUM_ROWS = 76
DIM = 64
B = 16384 * 200
NC, NS = 2, 16
NW = NC * NS
B_PER_W = B // NW
CHUNK = 128
N_CHUNKS = B_PER_W // CHUNK


def _emb_body(day_hbm, table_hbm, out_hbm, idx_v, rows_v, sem):
    wid = lax.axis_index("s") * NC + lax.axis_index("c")
    base = wid * B_PER_W

    def body(i, carry):
        cbase = base + i * CHUNK
        pltpu.sync_copy(day_hbm.at[pl.ds(cbase, CHUNK)], idx_v)
        pltpu.async_copy(table_hbm.at[idx_v], rows_v, sem).wait()
        pltpu.sync_copy(rows_v, out_hbm.at[pl.ds(cbase, CHUNK)])
        return carry

    lax.fori_loop(0, N_CHUNKS, body, 0)


@jax.jit
def _emb(day_flat, table):
    mesh = plsc.VectorSubcoreMesh(core_axis_name="c", subcore_axis_name="s")
    f = functools.partial(
        pl.kernel,
        out_type=jax.ShapeDtypeStruct((B, DIM), jnp.float32),
        mesh=mesh,
        scratch_types=[
            pltpu.VMEM((CHUNK,), jnp.int32),
            pltpu.VMEM((CHUNK, DIM), jnp.float32),
            pltpu.SemaphoreType.DMA,
        ],
    )(_emb_body)
    return f(day_flat, table)


def kernel(day, table):
    day_flat = day.reshape(B)
    out = _emb(day_flat, table)
    return out.reshape(day.shape[0], day.shape[1], DIM)


# speedup vs baseline: 2.6749x; 2.6749x over previous
import functools

import jax
import jax.numpy as jnp
from jax import lax
from jax.experimental import pallas as pl
from jax.experimental.pallas import tpu as pltpu
from jax.experimental.pallas import tpu_sc as plsc

NUM_ROWS = 76
DIM = 64
B = 16384 * 200
NC, NS = 2, 16
NW = NC * NS
B_PER_W = B // NW
CHUNK = 128
N_CHUNKS = B_PER_W // CHUNK


def _emb_body(day_hbm, table_hbm, out_hbm, idx_v, rows_v, sem):
    wid = lax.axis_index("s") * NC + lax.axis_index("c")
    base = wid * B_PER_W

    def body(i, carry):
        cbase = base + i * CHUNK
        pltpu.sync_copy(day_hbm.at[pl.ds(cbase, CHUNK)], idx_v)
        pltpu.async_copy(table_hbm.at[idx_v], rows_v, sem).wait()
        pltpu.sync_copy(rows_v, out_hbm.at[pl.ds(cbase, CHUNK)])
        return carry

    lax.fori_loop(0, N_CHUNKS, body, 0)


@jax.jit
def _emb(day_flat, table):
    mesh = plsc.VectorSubcoreMesh(core_axis_name="c", subcore_axis_name="s")
    f = functools.partial(
        pl.kernel,
        out_type=jax.ShapeDtypeStruct((B, DIM), jnp.float32),
        mesh=mesh,
        scratch_types=[
            pltpu.VMEM((CHUNK,), jnp.int32),
            pltpu.VMEM((CHUNK, DIM), jnp.float32),
            pltpu.SemaphoreType.DMA,
        ],
        compiler_params=pltpu.CompilerParams(use_tc_tiling_on_sc=False),
    )(_emb_body)
    return f(day_flat, table)


def kernel(day, table):
    day_flat = day.reshape(B)
    out = _emb(day_flat, table)
    return out.reshape(day.shape[0], day.shape[1], DIM)


# speedup vs baseline: 5.4292x; 2.0297x over previous
import functools

import jax
import jax.numpy as jnp
from jax import lax
from jax.experimental import pallas as pl
from jax.experimental.pallas import tpu as pltpu
from jax.experimental.pallas import tpu_sc as plsc

NUM_ROWS = 76
DIM = 64
B = 16384 * 200
NC, NS = 2, 16
NW = NC * NS
B_PER_W = B // NW
CHUNK = 128
NBUF = 4
N_OUTER = B_PER_W // (CHUNK * NBUF)


def _emb_body(day_hbm, table_hbm, out_hbm, tab_v, idx_v, rows_v,
              gsems, wsems):
    sid = lax.axis_index("s")
    wid = sid * NC + lax.axis_index("c")
    base = wid * B_PER_W

    @pl.when(sid == 0)
    def _():
        pltpu.sync_copy(table_hbm, tab_v)

    plsc.subcore_barrier()

    def chunk_step(o, b, first):
        cbase = base + (o * NBUF + b) * CHUNK
        if not first:
            pltpu.make_async_copy(
                rows_v.at[b], out_hbm.at[pl.ds(cbase, CHUNK)], wsems[b]
            ).wait()
        pltpu.async_copy(
            tab_v.at[idx_v.at[pl.ds(b * CHUNK, CHUNK)]],
            rows_v.at[b],
            gsems[b],
        ).wait()
        pltpu.async_copy(
            rows_v.at[b], out_hbm.at[pl.ds(cbase, CHUNK)], wsems[b]
        )

    def load_slab(o):
        pltpu.sync_copy(
            day_hbm.at[pl.ds(base + o * (NBUF * CHUNK), NBUF * CHUNK)], idx_v
        )

    load_slab(0)
    for b in range(NBUF):
        chunk_step(0, b, True)

    def outer(o, carry):
        load_slab(o)
        for b in range(NBUF):
            chunk_step(o, b, False)
        return carry

    lax.fori_loop(1, N_OUTER, outer, 0)

    for b in range(NBUF):
        cbase = base + ((N_OUTER - 1) * NBUF + b) * CHUNK
        pltpu.make_async_copy(
            rows_v.at[b], out_hbm.at[pl.ds(cbase, CHUNK)], wsems[b]
        ).wait()


@jax.jit
def _emb(day_flat, table):
    mesh = plsc.VectorSubcoreMesh(core_axis_name="c", subcore_axis_name="s")
    f = functools.partial(
        pl.kernel,
        out_type=jax.ShapeDtypeStruct((B, DIM), jnp.float32),
        mesh=mesh,
        scratch_types=[
            pltpu.VMEM_SHARED((NUM_ROWS, DIM), jnp.float32),
            pltpu.VMEM((NBUF * CHUNK,), jnp.int32),
            pltpu.VMEM((NBUF, CHUNK, DIM), jnp.float32),
            [pltpu.SemaphoreType.DMA] * NBUF,
            [pltpu.SemaphoreType.DMA] * NBUF,
        ],
        compiler_params=pltpu.CompilerParams(use_tc_tiling_on_sc=False),
    )(_emb_body)
    return f(day_flat, table)


def kernel(day, table):
    day_flat = day.reshape(B)
    out = _emb(day_flat, table)
    return out.reshape(day.shape[0], day.shape[1], DIM)


# speedup vs baseline: 5.6411x; 1.0390x over previous
import functools

import jax
import jax.numpy as jnp
from jax import lax
from jax.experimental import pallas as pl
from jax.experimental.pallas import tpu as pltpu
from jax.experimental.pallas import tpu_sc as plsc

NUM_ROWS = 76
DIM = 64
B = 16384 * 200
NC, NS = 2, 16
NW = NC * NS
B_PER_W = B // NW
CHUNK = 256
NBUF = 4
N_OUTER = B_PER_W // (CHUNK * NBUF)


def _emb_body(day_hbm, table_hbm, out_hbm, tab_v, idx_v, rows_v,
              gsems, wsems):
    sid = lax.axis_index("s")
    wid = sid * NC + lax.axis_index("c")
    base = wid * B_PER_W

    @pl.when(sid == 0)
    def _():
        pltpu.sync_copy(table_hbm, tab_v)

    plsc.subcore_barrier()

    def chunk_step(o, b, first):
        cbase = base + (o * NBUF + b) * CHUNK
        if not first:
            pltpu.make_async_copy(
                rows_v.at[b], out_hbm.at[pl.ds(cbase, CHUNK)], wsems[b]
            ).wait()
        pltpu.async_copy(
            tab_v.at[idx_v.at[pl.ds(b * CHUNK, CHUNK)]],
            rows_v.at[b],
            gsems[b],
        ).wait()
        pltpu.async_copy(
            rows_v.at[b], out_hbm.at[pl.ds(cbase, CHUNK)], wsems[b]
        )

    def load_slab(o):
        pltpu.sync_copy(
            day_hbm.at[pl.ds(base + o * (NBUF * CHUNK), NBUF * CHUNK)], idx_v
        )

    load_slab(0)
    for b in range(NBUF):
        chunk_step(0, b, True)

    def outer(o, carry):
        load_slab(o)
        for b in range(NBUF):
            chunk_step(o, b, False)
        return carry

    lax.fori_loop(1, N_OUTER, outer, 0)

    for b in range(NBUF):
        cbase = base + ((N_OUTER - 1) * NBUF + b) * CHUNK
        pltpu.make_async_copy(
            rows_v.at[b], out_hbm.at[pl.ds(cbase, CHUNK)], wsems[b]
        ).wait()


@jax.jit
def _emb(day_flat, table):
    mesh = plsc.VectorSubcoreMesh(core_axis_name="c", subcore_axis_name="s")
    f = functools.partial(
        pl.kernel,
        out_type=jax.ShapeDtypeStruct((B, DIM), jnp.float32),
        mesh=mesh,
        scratch_types=[
            pltpu.VMEM_SHARED((NUM_ROWS, DIM), jnp.float32),
            pltpu.VMEM((NBUF * CHUNK,), jnp.int32),
            pltpu.VMEM((NBUF, CHUNK, DIM), jnp.float32),
            [pltpu.SemaphoreType.DMA] * NBUF,
            [pltpu.SemaphoreType.DMA] * NBUF,
        ],
        compiler_params=pltpu.CompilerParams(use_tc_tiling_on_sc=False),
    )(_emb_body)
    return f(day_flat, table)


def kernel(day, table):
    day_flat = day.reshape(B)
    out = _emb(day_flat, table)
    return out.reshape(day.shape[0], day.shape[1], DIM)
